# baseline (speedup 1.0000x reference)
import functools

import jax
import jax.numpy as jnp
from jax.experimental import pallas as pl
from jax.experimental.pallas import tpu as pltpu

_SIZE = 16384
_SMOOTH = 0.1
_CONF = 0.9
_S = _SMOOTH / (_SIZE - 2)


def _ls_kernel(ts_ref, t_ref, x_ref, o_ref):
    xb = x_ref[...]
    tcol = t_ref[0]
    br = xb.shape[0]
    rowsum = jnp.sum(xb, axis=1, keepdims=True)
    x0 = xb[:, 0:1]
    k_const = _SMOOTH * jnp.log(_S) + _CONF * jnp.log(_CONF)
    contrib = jnp.where(tcol != 0, k_const - _S * rowsum + _S * x0, 0.0)

    lane = jax.lax.broadcasted_iota(jnp.int32, (1, 128), 1)
    acc = jnp.zeros((1, 128), jnp.float32)
    for r in range(br):
        t = ts_ref[0, r, 0]
        v = x_ref[pl.ds(r, 1), pl.ds((t >> 7) * 128, 128)]
        acc += jnp.where((lane == (t & 127)) & (t != 0), v, 0.0)

    total = jnp.sum(contrib) + (_S - _CONF) * jnp.sum(acc)
    o_ref[...] = total.reshape(1, 1, 1)


def _sum_kernel(p_ref, o_ref):
    o_ref[...] = jnp.sum(p_ref[...]).reshape(1, 1)


def kernel(x, target):
    n, c = x.shape
    br = 128
    n_blocks = n // br
    tr = target.reshape(n_blocks, br, 1)
    partials = pl.pallas_call(
        _ls_kernel,
        grid=(n_blocks,),
        in_specs=[
            pl.BlockSpec((1, br, 1), lambda i: (i, 0, 0),
                         memory_space=pltpu.SMEM),
            pl.BlockSpec((1, br, 1), lambda i: (i, 0, 0)),
            pl.BlockSpec((br, c), lambda i: (i, 0)),
        ],
        out_specs=pl.BlockSpec((1, 1, 1), lambda i: (i, 0, 0)),
        out_shape=jax.ShapeDtypeStruct((n_blocks, 1, 1), jnp.float32),
        compiler_params=pltpu.CompilerParams(
            dimension_semantics=("parallel",)),
    )(tr, tr, x)
    out = pl.pallas_call(
        _sum_kernel,
        in_specs=[pl.BlockSpec((n_blocks, 1, 1), lambda: (0, 0, 0))],
        out_specs=pl.BlockSpec((1, 1), lambda: (0, 0)),
        out_shape=jax.ShapeDtypeStruct((1, 1), jnp.float32),
    )(partials)
    return out[0, 0]


# speedup vs baseline: 1.0168x; 1.0168x over previous
import functools

import jax
import jax.numpy as jnp
from jax.experimental import pallas as pl
from jax.experimental.pallas import tpu as pltpu

_SIZE = 16384
_SMOOTH = 0.1
_CONF = 0.9
_S = _SMOOTH / (_SIZE - 2)


def _ls_kernel(ts_ref, t_ref, x_ref, o_ref, *, n_blocks):
    i = pl.program_id(0)
    xb = x_ref[...]
    tcol = t_ref[0]
    br = xb.shape[0]
    rowsum = jnp.sum(xb, axis=1, keepdims=True)
    x0 = xb[:, 0:1]
    k_const = _SMOOTH * jnp.log(_S) + _CONF * jnp.log(_CONF)
    contrib = jnp.where(tcol != 0, k_const - _S * rowsum + _S * x0, 0.0)

    lane = jax.lax.broadcasted_iota(jnp.int32, (1, 128), 1)
    acc = jnp.zeros((1, 128), jnp.float32)
    for r in range(br):
        t = ts_ref[0, r, 0]
        v = x_ref[pl.ds(r, 1), pl.ds((t >> 7) * 128, 128)]
        acc += jnp.where((lane == (t & 127)) & (t != 0), v, 0.0)

    total = (jnp.sum(contrib) + (_S - _CONF) * jnp.sum(acc)).reshape(1, 1)

    @pl.when(i == 0)
    def _():
        o_ref[...] = jnp.zeros_like(o_ref)

    o_ref[...] += total


def kernel(x, target):
    n, c = x.shape
    br = 128
    n_blocks = n // br
    tr = target.reshape(n_blocks, br, 1)
    out = pl.pallas_call(
        functools.partial(_ls_kernel, n_blocks=n_blocks),
        grid=(n_blocks,),
        in_specs=[
            pl.BlockSpec((1, br, 1), lambda i: (i, 0, 0),
                         memory_space=pltpu.SMEM),
            pl.BlockSpec((1, br, 1), lambda i: (i, 0, 0)),
            pl.BlockSpec((br, c), lambda i: (i, 0)),
        ],
        out_specs=pl.BlockSpec((1, 1), lambda i: (0, 0)),
        out_shape=jax.ShapeDtypeStruct((1, 1), jnp.float32),
    )(tr, tr, x)
    return out[0, 0]
